# baseline (speedup 1.0000x reference)
import jax
import jax.numpy as jnp
from jax.experimental import pallas as pl

_B = 2048
_N = 64
_H = 128
_FEAT = 512
_FH = _FEAT // 2
_BB = 64


def _tc_body(it_ref, ii_ref, xi_ref, xt_ref, wu_ref, bu_ref,
             wli_ref, wlt_ref, wri_ref, wrt_ref, bli_ref, blt_ref, out_ref):
    inv_n = jnp.float32(1.0 / _N)
    agg_i = jnp.sum(xi_ref[...], axis=1) * inv_n
    agg_t = jnp.sum(xt_ref[...], axis=1) * inv_n
    dn = (((1,), (1,)), ((), ()))
    it = it_ref[...].reshape(_BB, _FH)
    ii = ii_ref[...].reshape(_BB, _FH)
    wu = wu_ref[...]
    user = jax.lax.dot_general(it, wu[:, :_FH], dn,
                               preferred_element_type=jnp.float32)
    user = user + jax.lax.dot_general(ii, wu[:, _FH:], dn,
                                      preferred_element_type=jnp.float32)
    user = user + bu_ref[...]
    acc = jax.lax.dot_general(agg_i, wli_ref[...], dn,
                              preferred_element_type=jnp.float32)
    acc = acc + jax.lax.dot_general(agg_t, wlt_ref[...], dn,
                                    preferred_element_type=jnp.float32)
    wr = wri_ref[...] + wrt_ref[...]
    acc = acc + jax.lax.dot_general(user, wr, dn,
                                    preferred_element_type=jnp.float32)
    out_ref[...] = jnp.maximum(acc + bli_ref[...] + blt_ref[...], 0.0)


@jax.jit
def kernel(input_text, input_img, base_text_features, base_img_features,
           W_user, b_user, Wl_img, bl_img, Wr_img, Wl_txt, bl_txt, Wr_txt):
    grid = (_B // _BB,)
    full = lambda shape: pl.BlockSpec(shape, lambda i: (0,) * len(shape))
    out = pl.pallas_call(
        _tc_body,
        grid=grid,
        in_specs=[
            pl.BlockSpec((_BB, 1, _FH), lambda i: (i, 0, 0)),
            pl.BlockSpec((_BB, 1, _FH), lambda i: (i, 0, 0)),
            pl.BlockSpec((_BB, _N, _H), lambda i: (i, 0, 0)),
            pl.BlockSpec((_BB, _N, _H), lambda i: (i, 0, 0)),
            full((_H, _FEAT)),
            full((1, _H)),
            full((_H, _H)),
            full((_H, _H)),
            full((_H, _H)),
            full((_H, _H)),
            full((1, _H)),
            full((1, _H)),
        ],
        out_specs=pl.BlockSpec((_BB, _H), lambda i: (i, 0)),
        out_shape=jax.ShapeDtypeStruct((_B, _H), jnp.float32),
    )(input_text, input_img, base_img_features, base_text_features,
      W_user, b_user.reshape(1, _H), Wl_img, Wl_txt, Wr_img, Wr_txt,
      bl_img.reshape(1, _H), bl_txt.reshape(1, _H))
    return out


# speedup vs baseline: 1.1184x; 1.1184x over previous
import jax
import jax.numpy as jnp
from jax.experimental import pallas as pl

_B = 2048
_N = 64
_H = 128
_FEAT = 512
_FH = _FEAT // 2
_BB = 128


def _tc_body(it_ref, ii_ref, xi_ref, xt_ref, wu_ref, bu_ref,
             wli_ref, wlt_ref, wri_ref, wrt_ref, bli_ref, blt_ref, out_ref):
    inv_n = jnp.float32(1.0 / _N)
    agg_i = jnp.sum(xi_ref[...], axis=1) * inv_n
    agg_t = jnp.sum(xt_ref[...], axis=1) * inv_n
    dn = (((1,), (1,)), ((), ()))
    it = it_ref[...].reshape(_BB, _FH)
    ii = ii_ref[...].reshape(_BB, _FH)
    wu = wu_ref[...]
    user = jax.lax.dot_general(it, wu[:, :_FH], dn,
                               preferred_element_type=jnp.float32)
    user = user + jax.lax.dot_general(ii, wu[:, _FH:], dn,
                                      preferred_element_type=jnp.float32)
    user = user + bu_ref[...]
    acc = jax.lax.dot_general(agg_i, wli_ref[...], dn,
                              preferred_element_type=jnp.float32)
    acc = acc + jax.lax.dot_general(agg_t, wlt_ref[...], dn,
                                    preferred_element_type=jnp.float32)
    wr = wri_ref[...] + wrt_ref[...]
    acc = acc + jax.lax.dot_general(user, wr, dn,
                                    preferred_element_type=jnp.float32)
    out_ref[...] = jnp.maximum(acc + bli_ref[...] + blt_ref[...], 0.0)


@jax.jit
def kernel(input_text, input_img, base_text_features, base_img_features,
           W_user, b_user, Wl_img, bl_img, Wr_img, Wl_txt, bl_txt, Wr_txt):
    grid = (_B // _BB,)
    full = lambda shape: pl.BlockSpec(shape, lambda i: (0,) * len(shape))
    out = pl.pallas_call(
        _tc_body,
        grid=grid,
        in_specs=[
            pl.BlockSpec((_BB, 1, _FH), lambda i: (i, 0, 0)),
            pl.BlockSpec((_BB, 1, _FH), lambda i: (i, 0, 0)),
            pl.BlockSpec((_BB, _N, _H), lambda i: (i, 0, 0)),
            pl.BlockSpec((_BB, _N, _H), lambda i: (i, 0, 0)),
            full((_H, _FEAT)),
            full((1, _H)),
            full((_H, _H)),
            full((_H, _H)),
            full((_H, _H)),
            full((_H, _H)),
            full((1, _H)),
            full((1, _H)),
        ],
        out_specs=pl.BlockSpec((_BB, _H), lambda i: (i, 0)),
        out_shape=jax.ShapeDtypeStruct((_B, _H), jnp.float32),
    )(input_text, input_img, base_img_features, base_text_features,
      W_user, b_user.reshape(1, _H), Wl_img, Wl_txt, Wr_img, Wr_txt,
      bl_img.reshape(1, _H), bl_txt.reshape(1, _H))
    return out
